# baseline (speedup 1.0000x reference)
import math

import jax
import jax.numpy as jnp
from jax.experimental import pallas as pl
from jax.experimental.pallas import tpu as pltpu

PAD = 1
S = 2048
W = S + 16


def _cumsum_lanes(x, n):
    k = 1
    while k < n:
        shifted = jnp.concatenate(
            [jnp.zeros((1, k), x.dtype), x[:, : n - k]], axis=1)
        x = x + shifted
        k *= 2
    return x


def _body(c0s_ref, npads_ref, ids_ref, freqs_ref, ci_ref, si_ref, w_hbm,
          out_ref, win, sem):
    t = pl.program_id(0)
    c0 = c0s_ref[t]
    npad = npads_ref[t]
    half = ci_ref.shape[1]

    @pl.when(npad == 0)
    def _fast():
        p0f = (c0 + 2).astype(jnp.float32) * freqs_ref[...]
        s0 = jnp.sin(p0f)
        cs0 = jnp.cos(p0f)
        ci = ci_ref[...]
        si = si_ref[...]
        out_ref[:, :half] = s0 * ci + cs0 * si
        out_ref[:, half:] = cs0 * ci - s0 * si

    @pl.when(npad != 0)
    def _slow():
        a0 = ((c0 + 1) // 8) * 8
        r = (c0 + 1) - a0
        cp = pltpu.make_async_copy(w_hbm.at[pl.ds(a0, W), :], win, sem)
        cp.start()
        ids = ids_ref[0]
        bmask = (ids != PAD).astype(jnp.int32)
        lcum = _cumsum_lanes(bmask, S)
        local = lcum * bmask - c0 * (1 - bmask)
        oh = (jax.lax.broadcasted_iota(jnp.int32, (W, S), 0)
              == local + r).astype(jnp.bfloat16)
        cp.wait()
        out_ref[...] = jax.lax.dot_general(
            oh, win[...].astype(jnp.bfloat16),
            dimension_numbers=(((0,), (0,)), ((), ())),
            preferred_element_type=jnp.float32,
        )


@jax.jit
def kernel(input, weights):
    bsz, seq = input.shape
    nrows, d = weights.shape
    nb = seq // S
    half = d // 2

    p_rows = ((seq - S) + 1 + W + 7) // 8 * 8
    w_pad = jnp.pad(weights, ((0, p_rows - nrows), (0, 0)))

    emb = math.log(10000.0) / (half - 1)
    freqs = jnp.exp(jnp.arange(half, dtype=jnp.float32) * -emb)[None, :]
    iang = jnp.arange(S, dtype=jnp.float32)[:, None] * freqs
    ci, si = jnp.cos(iang), jnp.sin(iang)

    mask = (input != PAD).astype(jnp.int32)
    blk = mask.reshape(bsz, nb, S).sum(axis=-1)
    c0s = (jnp.cumsum(blk, axis=1) - blk).reshape(-1)
    npads = (S - blk).reshape(-1)
    ids3 = input.reshape(bsz * nb, 1, S)

    out = pl.pallas_call(
        _body,
        grid_spec=pltpu.PrefetchScalarGridSpec(
            num_scalar_prefetch=2,
            grid=(bsz * nb,),
            in_specs=[
                pl.BlockSpec((1, 1, S), lambda t, c, p: (t, 0, 0)),
                pl.BlockSpec((1, half), lambda t, c, p: (0, 0)),
                pl.BlockSpec((S, half), lambda t, c, p: (0, 0)),
                pl.BlockSpec((S, half), lambda t, c, p: (0, 0)),
                pl.BlockSpec(memory_space=pl.ANY),
            ],
            out_specs=pl.BlockSpec((S, d), lambda t, c, p: (t, 0)),
            scratch_shapes=[
                pltpu.VMEM((W, d), jnp.float32),
                pltpu.SemaphoreType.DMA,
            ],
        ),
        out_shape=jax.ShapeDtypeStruct((bsz * seq, d), jnp.float32),
    )(c0s, npads, ids3, freqs, ci, si, w_pad)
    return out.reshape(bsz, seq, d)


# speedup vs baseline: 1.1118x; 1.1118x over previous
import math

import jax
import jax.numpy as jnp
from jax.experimental import pallas as pl
from jax.experimental.pallas import tpu as pltpu

PAD = 1
S = 512
W = S + 16


def _cumsum_lanes(x, n):
    k = 1
    while k < n:
        shifted = jnp.concatenate(
            [jnp.zeros((1, k), x.dtype), x[:, : n - k]], axis=1)
        x = x + shifted
        k *= 2
    return x


def _body(c0s_ref, npads_ref, ids_ref, freqs_ref, ci_ref, si_ref, w_hbm,
          out_ref, win, sem):
    t = pl.program_id(0)
    c0 = c0s_ref[t]
    npad = npads_ref[t]
    half = ci_ref.shape[1]

    @pl.when(npad == 0)
    def _fast():
        p0f = (c0 + 2).astype(jnp.float32) * freqs_ref[...]
        s0 = jnp.sin(p0f)
        cs0 = jnp.cos(p0f)
        ci = ci_ref[...]
        si = si_ref[...]
        out_ref[:, :half] = s0 * ci + cs0 * si
        out_ref[:, half:] = cs0 * ci - s0 * si

    @pl.when(npad != 0)
    def _slow():
        a0 = ((c0 + 1) // 8) * 8
        r = (c0 + 1) - a0
        cp = pltpu.make_async_copy(w_hbm.at[pl.ds(a0, W), :], win, sem)
        cp.start()
        ids = ids_ref[0]
        bmask = (ids != PAD).astype(jnp.int32)
        lcum = _cumsum_lanes(bmask, S)
        local = lcum * bmask - c0 * (1 - bmask)
        oh = (jax.lax.broadcasted_iota(jnp.int32, (W, S), 0)
              == local + r).astype(jnp.bfloat16)
        cp.wait()
        out_ref[...] = jax.lax.dot_general(
            oh, win[...].astype(jnp.bfloat16),
            dimension_numbers=(((0,), (0,)), ((), ())),
            preferred_element_type=jnp.float32,
        )


@jax.jit
def kernel(input, weights):
    bsz, seq = input.shape
    nrows, d = weights.shape
    nb = seq // S
    half = d // 2

    p_rows = ((seq - S) + 1 + W + 7) // 8 * 8
    w_pad = jnp.pad(weights, ((0, p_rows - nrows), (0, 0)))

    emb = math.log(10000.0) / (half - 1)
    freqs = jnp.exp(jnp.arange(half, dtype=jnp.float32) * -emb)[None, :]
    iang = jnp.arange(S, dtype=jnp.float32)[:, None] * freqs
    ci, si = jnp.cos(iang), jnp.sin(iang)

    mask = (input != PAD).astype(jnp.int32)
    blk = mask.reshape(bsz, nb, S).sum(axis=-1)
    c0s = (jnp.cumsum(blk, axis=1) - blk).reshape(-1)
    npads = (S - blk).reshape(-1)
    ids3 = input.reshape(bsz * nb, 1, S)

    out = pl.pallas_call(
        _body,
        grid_spec=pltpu.PrefetchScalarGridSpec(
            num_scalar_prefetch=2,
            grid=(bsz * nb,),
            in_specs=[
                pl.BlockSpec((1, 1, S), lambda t, c, p: (t, 0, 0)),
                pl.BlockSpec((1, half), lambda t, c, p: (0, 0)),
                pl.BlockSpec((S, half), lambda t, c, p: (0, 0)),
                pl.BlockSpec((S, half), lambda t, c, p: (0, 0)),
                pl.BlockSpec(memory_space=pl.ANY),
            ],
            out_specs=pl.BlockSpec((S, d), lambda t, c, p: (t, 0)),
            scratch_shapes=[
                pltpu.VMEM((W, d), jnp.float32),
                pltpu.SemaphoreType.DMA,
            ],
        ),
        out_shape=jax.ShapeDtypeStruct((bsz * seq, d), jnp.float32),
    )(c0s, npads, ids3, freqs, ci, si, w_pad)
    return out.reshape(bsz, seq, d)


# speedup vs baseline: 1.2078x; 1.0864x over previous
import math

import jax
import jax.numpy as jnp
from jax.experimental import pallas as pl
from jax.experimental.pallas import tpu as pltpu

PAD = 1
S = 1024
W = S + 16


def _cumsum_lanes(x, n):
    k = 1
    while k < n:
        shifted = jnp.concatenate(
            [jnp.zeros((1, k), x.dtype), x[:, : n - k]], axis=1)
        x = x + shifted
        k *= 2
    return x


def _body(c0s_ref, npads_ref, ids_ref, freqs_ref, ci_ref, si_ref, w_hbm,
          out_ref, win, sem):
    t = pl.program_id(0)
    c0 = c0s_ref[t]
    npad = npads_ref[t]
    half = ci_ref.shape[1]

    @pl.when(npad == 0)
    def _fast():
        p0f = (c0 + 2).astype(jnp.float32) * freqs_ref[...]
        s0 = jnp.sin(p0f)
        cs0 = jnp.cos(p0f)
        ci = ci_ref[...]
        si = si_ref[...]
        out_ref[:, :half] = s0 * ci + cs0 * si
        out_ref[:, half:] = cs0 * ci - s0 * si

    @pl.when(npad != 0)
    def _slow():
        a0 = ((c0 + 1) // 8) * 8
        r = (c0 + 1) - a0
        cp = pltpu.make_async_copy(w_hbm.at[pl.ds(a0, W), :], win, sem)
        cp.start()
        ids = ids_ref[0]
        bmask = (ids != PAD).astype(jnp.int32)
        lcum = _cumsum_lanes(bmask, S)
        local = lcum * bmask - c0 * (1 - bmask)
        oh = (jax.lax.broadcasted_iota(jnp.int32, (W, S), 0)
              == local + r).astype(jnp.bfloat16)
        cp.wait()
        out_ref[...] = jax.lax.dot_general(
            oh, win[...].astype(jnp.bfloat16),
            dimension_numbers=(((0,), (0,)), ((), ())),
            preferred_element_type=jnp.float32,
        )


@jax.jit
def kernel(input, weights):
    bsz, seq = input.shape
    nrows, d = weights.shape
    nb = seq // S
    half = d // 2

    p_rows = ((seq - S) + 1 + W + 7) // 8 * 8
    w_pad = jnp.pad(weights, ((0, p_rows - nrows), (0, 0)))

    emb = math.log(10000.0) / (half - 1)
    freqs = jnp.exp(jnp.arange(half, dtype=jnp.float32) * -emb)[None, :]
    iang = jnp.arange(S, dtype=jnp.float32)[:, None] * freqs
    ci, si = jnp.cos(iang), jnp.sin(iang)

    mask = (input != PAD).astype(jnp.int32)
    blk = mask.reshape(bsz, nb, S).sum(axis=-1)
    c0s = (jnp.cumsum(blk, axis=1) - blk).reshape(-1)
    npads = (S - blk).reshape(-1)
    ids3 = input.reshape(bsz * nb, 1, S)

    out = pl.pallas_call(
        _body,
        grid_spec=pltpu.PrefetchScalarGridSpec(
            num_scalar_prefetch=2,
            grid=(bsz * nb,),
            in_specs=[
                pl.BlockSpec((1, 1, S), lambda t, c, p: (t, 0, 0)),
                pl.BlockSpec((1, half), lambda t, c, p: (0, 0)),
                pl.BlockSpec((S, half), lambda t, c, p: (0, 0)),
                pl.BlockSpec((S, half), lambda t, c, p: (0, 0)),
                pl.BlockSpec(memory_space=pl.ANY),
            ],
            out_specs=pl.BlockSpec((S, d), lambda t, c, p: (t, 0)),
            scratch_shapes=[
                pltpu.VMEM((W, d), jnp.float32),
                pltpu.SemaphoreType.DMA,
            ],
        ),
        out_shape=jax.ShapeDtypeStruct((bsz * seq, d), jnp.float32),
        compiler_params=pltpu.CompilerParams(
            dimension_semantics=(pltpu.PARALLEL,)),
    )(c0s, npads, ids3, freqs, ci, si, w_pad)
    return out.reshape(bsz, seq, d)
